# baseline (speedup 1.0000x reference)
import functools

import jax
import jax.numpy as jnp
from jax import lax
from jax.experimental import pallas as pl
from jax.experimental.pallas import tpu as pltpu
from jax.experimental.pallas import tpu_sc as plsc

_EPC = 2
_NBUF = 4


@functools.lru_cache(maxsize=None)
def _build_gather(batch: int, steps: int, D: int):
    info = plsc.get_sparse_core_info()
    nc, ns = info.num_cores, info.num_subcores
    nw = nc * ns
    e_per_w = batch // nw
    C = _EPC * steps
    n_chunks = e_per_w // _EPC
    n_periods = n_chunks // _NBUF
    assert C <= 128 and batch % nw == 0 and e_per_w % _EPC == 0
    assert n_chunks % _NBUF == 0

    mesh = plsc.VectorSubcoreMesh(core_axis_name="c", subcore_axis_name="s")

    def body(table_hbm, idx_hbm, out_hbm, idx_v, rows_v, gsem, *wsems):
        wid = lax.axis_index("s") * nc + lax.axis_index("c")
        ebase = wid * e_per_w
        pltpu.sync_copy(idx_hbm.at[wid], idx_v)

        def wait_writeout(b):
            for e in range(_EPC):
                pltpu.make_async_copy(
                    rows_v.at[b, pl.ds(e * steps, steps)],
                    out_hbm.at[ebase], wsems[b]).wait()

        def period(o, carry):
            descs = []
            for b in range(_NBUF):
                j = o * _NBUF + b

                @pl.when(o > 0)
                def _():
                    wait_writeout(b)

                descs.append(pltpu.async_copy(
                    table_hbm.at[idx_v.at[j]], rows_v.at[b], gsem))
            for b in range(_NBUF):
                j = o * _NBUF + b
                descs[b].wait()
                for e in range(_EPC):
                    pltpu.async_copy(
                        rows_v.at[b, pl.ds(e * steps, steps)],
                        out_hbm.at[ebase + j * _EPC + e], wsems[b])
            return carry

        lax.fori_loop(0, n_periods, period, 0)
        for b in range(_NBUF):
            wait_writeout(b)

    return pl.kernel(
        body,
        mesh=mesh,
        compiler_params=pltpu.CompilerParams(use_tc_tiling_on_sc=True, needs_layout_passes=True),
        out_type=jax.ShapeDtypeStruct((batch, steps, D), jnp.float32),
        scratch_types=[
            pltpu.VMEM((n_chunks, C), jnp.int32),
            pltpu.VMEM((_NBUF, C, D), jnp.float32),
            pltpu.SemaphoreType.DMA,
        ] + [pltpu.SemaphoreType.DMA] * _NBUF,
    )


def kernel(inputs, embedding):
    batch, steps = inputs.shape
    d = embedding.shape[1]
    info = plsc.get_sparse_core_info()
    nw = info.num_cores * info.num_subcores
    n_chunks = batch // (nw * _EPC)
    idx = inputs.astype(jnp.int32).reshape(nw, n_chunks, _EPC * steps)
    return _build_gather(batch, steps, d)(embedding, idx)


# speedup vs baseline: 1.7450x; 1.7450x over previous
import functools

import jax
import jax.numpy as jnp
from jax import lax
from jax.experimental import pallas as pl
from jax.experimental.pallas import tpu as pltpu
from jax.experimental.pallas import tpu_sc as plsc

_NBUF = 5


@functools.lru_cache(maxsize=None)
def _build_gather(batch: int, steps: int, D: int):
    info = plsc.get_sparse_core_info()
    nc, ns = info.num_cores, info.num_subcores
    nw = nc * ns
    C = batch // nw
    n_chunks = steps
    n_periods = n_chunks // _NBUF
    assert C <= 128 and batch % nw == 0 and n_chunks % _NBUF == 0

    mesh = plsc.VectorSubcoreMesh(core_axis_name="c", subcore_axis_name="s")

    def body(table_hbm, idx_hbm, out_hbm, idx_v, rows_v, gsem, *wsems):
        wid = lax.axis_index("s") * nc + lax.axis_index("c")
        bbase = wid * C
        pltpu.sync_copy(idx_hbm.at[wid], idx_v)

        def wait_writeout(b):
            pltpu.make_async_copy(
                rows_v.at[b], out_hbm.at[0, pl.ds(bbase, C)], wsems[b]).wait()

        def period(o, carry):
            descs = []
            for b in range(_NBUF):
                j = o * _NBUF + b

                @pl.when(o > 0)
                def _():
                    wait_writeout(b)

                descs.append(pltpu.async_copy(
                    table_hbm.at[idx_v.at[j]], rows_v.at[b], gsem))
            for b in range(_NBUF):
                j = o * _NBUF + b
                descs[b].wait()
                pltpu.async_copy(
                    rows_v.at[b], out_hbm.at[j, pl.ds(bbase, C)], wsems[b])
            return carry

        lax.fori_loop(0, n_periods, period, 0)
        for b in range(_NBUF):
            wait_writeout(b)

    return pl.kernel(
        body,
        mesh=mesh,
        out_type=jax.ShapeDtypeStruct((steps, batch, D), jnp.float32),
        scratch_types=[
            pltpu.VMEM((n_chunks, C), jnp.int32),
            pltpu.VMEM((_NBUF, C, D), jnp.float32),
            pltpu.SemaphoreType.DMA,
        ] + [pltpu.SemaphoreType.DMA] * _NBUF,
    )


def kernel(inputs, embedding):
    batch, steps = inputs.shape
    d = embedding.shape[1]
    info = plsc.get_sparse_core_info()
    nw = info.num_cores * info.num_subcores
    c = batch // nw
    idx = inputs.astype(jnp.int32).T.reshape(steps, nw, c).transpose(1, 0, 2)
    out = _build_gather(batch, steps, d)(embedding, idx)
    return out.transpose(1, 0, 2)
